# Initial kernel scaffold; baseline (speedup 1.0000x reference)
import jax
import jax.numpy as jnp
from jax.experimental import pallas as pl


def kernel(pointcloud, params):
    raise NotImplementedError("write your pallas kernel here")


# speedup vs baseline: 2.9185x; 2.9185x over previous
import functools

import jax
import jax.numpy as jnp
import numpy as np
from jax.experimental import pallas as pl
from jax.experimental.pallas import tpu as pltpu

_SA_CFG = [
    (4096, [0.1, 0.5], [16, 32], [[4, 16, 16, 32], [4, 32, 32, 64]]),
    (1024, [0.5, 1.0], [16, 32], [[99, 64, 64, 128], [99, 64, 96, 128]]),
    (256, [1.0, 2.0], [16, 32], [[259, 128, 196, 256], [259, 128, 196, 256]]),
    (64, [2.0, 4.0], [16, 32], [[515, 256, 256, 512], [515, 256, 384, 512]]),
]


def _fps_kernel_body(xs_ref, ys_ref, zs_ref, ox_ref, oy_ref, oz_ref, dist_ref,
                     *, npoint: int):
    S, L = xs_ref.shape
    N = S * L
    pos = (jax.lax.broadcasted_iota(jnp.int32, (S, L), 0) * L
           + jax.lax.broadcasted_iota(jnp.int32, (S, L), 1))
    dist_ref[:, :] = jnp.full((S, L), 1e10, jnp.float32)

    def body(i, far):
        m = pos == far
        cx = jnp.sum(jnp.where(m, xs_ref[:, :], 0.0))
        cy = jnp.sum(jnp.where(m, ys_ref[:, :], 0.0))
        cz = jnp.sum(jnp.where(m, zs_ref[:, :], 0.0))
        ox_ref[pl.ds(i, 1), :] = jnp.full((1, 1), cx, jnp.float32)
        oy_ref[pl.ds(i, 1), :] = jnp.full((1, 1), cy, jnp.float32)
        oz_ref[pl.ds(i, 1), :] = jnp.full((1, 1), cz, jnp.float32)
        dx = xs_ref[:, :] - cx
        dy = ys_ref[:, :] - cy
        dz = zs_ref[:, :] - cz
        d = dx * dx + dy * dy + dz * dz
        nd = jnp.minimum(dist_ref[:, :], d)
        dist_ref[:, :] = nd
        mx = jnp.max(nd)
        far2 = jnp.min(jnp.where(nd == mx, pos, N)).astype(jnp.int32)
        return far2

    jax.lax.fori_loop(0, npoint, body, jnp.int32(0), unroll=False)


def _fps_pallas(xyz, npoint):
    N = xyz.shape[0]
    S = 8
    L = N // S
    xs = xyz[:, 0].reshape(S, L)
    ys = xyz[:, 1].reshape(S, L)
    zs = xyz[:, 2].reshape(S, L)
    out_sds = jax.ShapeDtypeStruct((npoint, 1), jnp.float32)
    ox, oy, oz = pl.pallas_call(
        functools.partial(_fps_kernel_body, npoint=npoint),
        out_shape=(out_sds, out_sds, out_sds),
        scratch_shapes=[pltpu.VMEM((S, L), jnp.float32)],
    )(xs, ys, zs)
    return jnp.concatenate([ox, oy, oz], axis=1)


def _bn_fold(layers):
    out = []
    for (W, b, g, beta) in layers:
        s = g / jnp.sqrt(1.0 + 1e-5)
        out.append((W * s[None, :], b * s + beta))
    return out


def _sqdist2(src, dst):
    d = -2.0 * jnp.matmul(src, dst.T)
    d = d + jnp.sum(src * src, axis=-1)[:, None]
    d = d + jnp.sum(dst * dst, axis=-1)[None, :]
    return d


def _ball_query2(radius, nsample, xyz, new_xyz):
    N = xyz.shape[0]
    sqd = _sqdist2(new_xyz, xyz)
    mask = sqd <= radius * radius
    scores = jnp.where(mask, -jnp.arange(N, dtype=jnp.float32), -jnp.inf)
    vals, idx = jax.lax.top_k(scores, nsample)
    valid = vals > -jnp.inf
    idx = jnp.where(valid, idx, idx[:, 0:1])
    return idx


def _sa_msg2(xyz, features, npoint, radii, nsamples, scale_params):
    new_xyz = _fps_pallas(xyz, npoint)
    feats_t = None if features is None else features.T
    outs = []
    for radius, ns, layers in zip(radii, nsamples, scale_params):
        idx = _ball_query2(radius, ns, xyz, new_xyz)
        grouped_xyz = xyz[idx] - new_xyz[:, None, :]
        if feats_t is not None:
            grouped = jnp.concatenate([grouped_xyz, feats_t[idx]], axis=-1)
        else:
            grouped = grouped_xyz
        h = grouped
        for (W, b) in _bn_fold(layers):
            h = jax.nn.relu(jnp.matmul(h, W) + b)
        outs.append(jnp.max(h, axis=1))
    return new_xyz, jnp.concatenate(outs, axis=-1).T


def _fp2(unknown, known, unknown_feats, known_feats, layers):
    sqd = _sqdist2(unknown, known)
    vals, idx = jax.lax.top_k(-sqd, 3)
    dist = -vals
    dist_recip = 1.0 / (dist + 1e-8)
    weight = dist_recip / jnp.sum(dist_recip, axis=1, keepdims=True)
    kf = known_feats.T
    interp = jnp.sum(kf[idx] * weight[..., None], axis=1)
    if unknown_feats is not None:
        new = jnp.concatenate([interp, unknown_feats.T], axis=-1)
    else:
        new = interp
    h = new
    for (W, b) in _bn_fold(layers):
        h = jax.nn.relu(jnp.matmul(h, W) + b)
    return h.T


def kernel(pointcloud, params):
    xyz = pointcloud[0, :, 0:3]
    features = pointcloud[0, :, 3:].T
    l_xyz = [xyz]
    l_feats = [features]
    for k in range(4):
        npoint, radii, nsamples, _ = _SA_CFG[k]
        nx, nf = _sa_msg2(l_xyz[k], l_feats[k], npoint, radii, nsamples,
                          params["sa"][k])
        l_xyz.append(nx)
        l_feats.append(nf)
    for i in range(-1, -5, -1):
        l_feats[i - 1] = _fp2(l_xyz[i - 1], l_xyz[i], l_feats[i - 1],
                              l_feats[i], params["fp"][i])
    point_features = l_feats[0]
    return jnp.max(point_features, axis=-1)


# speedup vs baseline: 14.1680x; 4.8545x over previous
import functools

import jax
import jax.numpy as jnp
import numpy as np
from jax.experimental import pallas as pl
from jax.experimental.pallas import tpu as pltpu

_SA_CFG = [
    (4096, [0.1, 0.5], [16, 32], [[4, 16, 16, 32], [4, 32, 32, 64]]),
    (1024, [0.5, 1.0], [16, 32], [[99, 64, 64, 128], [99, 64, 96, 128]]),
    (256, [1.0, 2.0], [16, 32], [[259, 128, 196, 256], [259, 128, 196, 256]]),
    (64, [2.0, 4.0], [16, 32], [[515, 256, 256, 512], [515, 256, 384, 512]]),
]


def _fps_kernel_body(xs_ref, ys_ref, zs_ref, ox_ref, oy_ref, oz_ref, dist_ref,
                     *, npoint: int):
    S, L = xs_ref.shape
    N = S * L
    pos = (jax.lax.broadcasted_iota(jnp.int32, (S, L), 0) * L
           + jax.lax.broadcasted_iota(jnp.int32, (S, L), 1))
    dist_ref[:, :] = jnp.full((S, L), 1e10, jnp.float32)

    def body(i, far):
        m = pos == far
        cx = jnp.sum(jnp.where(m, xs_ref[:, :], 0.0))
        cy = jnp.sum(jnp.where(m, ys_ref[:, :], 0.0))
        cz = jnp.sum(jnp.where(m, zs_ref[:, :], 0.0))
        ox_ref[pl.ds(i, 1), :] = jnp.full((1, 1), cx, jnp.float32)
        oy_ref[pl.ds(i, 1), :] = jnp.full((1, 1), cy, jnp.float32)
        oz_ref[pl.ds(i, 1), :] = jnp.full((1, 1), cz, jnp.float32)
        dx = xs_ref[:, :] - cx
        dy = ys_ref[:, :] - cy
        dz = zs_ref[:, :] - cz
        d = dx * dx + dy * dy + dz * dz
        nd = jnp.minimum(dist_ref[:, :], d)
        dist_ref[:, :] = nd
        mx = jnp.max(nd)
        far2 = jnp.min(jnp.where(nd == mx, pos, N)).astype(jnp.int32)
        return far2

    jax.lax.fori_loop(0, npoint, body, jnp.int32(0), unroll=False)


def _fps_pallas(xyz, npoint):
    N = xyz.shape[0]
    S = 8
    L = N // S
    xs = xyz[:, 0].reshape(S, L)
    ys = xyz[:, 1].reshape(S, L)
    zs = xyz[:, 2].reshape(S, L)
    out_sds = jax.ShapeDtypeStruct((npoint, 1), jnp.float32)
    ox, oy, oz = pl.pallas_call(
        functools.partial(_fps_kernel_body, npoint=npoint),
        out_shape=(out_sds, out_sds, out_sds),
        scratch_shapes=[pltpu.VMEM((S, L), jnp.float32)],
    )(xs, ys, zs)
    return jnp.concatenate([ox, oy, oz], axis=1)


def _sa_scale_body(centers_ref, pointsT_ref, pxf_ref, w1_ref, w1p_ref,
                   b1_ref, w2_ref, b2_ref, w3_ref, b3_ref, out_ref,
                   *, ns: int, r2: float, M: int, N: int):
    f32 = jnp.float32
    t = jnp.dot(pxf_ref[:, :], w1_ref[:, :],
                preferred_element_type=f32) + b1_ref[:, :]
    cblk = centers_ref[:, :]
    pT = pointsT_ref[:, :]
    dots = jnp.dot(cblk, pT, preferred_element_type=f32)
    cn2 = jnp.sum(cblk * cblk, axis=1, keepdims=True)
    pn2 = jnp.sum(pT * pT, axis=0, keepdims=True)
    sqd = (-2.0 * dots + cn2) + pn2
    m01 = (sqd <= r2).astype(f32)
    triu = (jax.lax.broadcasted_iota(jnp.int32, (128, 128), 0)
            <= jax.lax.broadcasted_iota(jnp.int32, (128, 128), 1)).astype(f32)
    parts = []
    carry = jnp.zeros((M, 1), f32)
    for b in range(N // 128):
        rb = jnp.dot(m01[:, b * 128:(b + 1) * 128], triu,
                     preferred_element_type=f32) + carry
        parts.append(rb)
        carry = rb[:, 127:128]
    rank = jnp.concatenate(parts, axis=1)
    count = carry
    s = jnp.where((m01 > 0.0) & (rank <= float(ns)), rank, 0.0)
    gs = []
    for j in range(ns):
        ohj = (s == float(j + 1)).astype(f32)
        gs.append(jnp.dot(ohj, t, preferred_element_type=f32))
    g = jnp.concatenate(gs, axis=0)
    u = jnp.dot(cblk, w1p_ref[:, :], preferred_element_type=f32)
    urep = jnp.concatenate([u] * ns, axis=0)
    h = jnp.maximum(g - urep, 0.0)
    h = jnp.maximum(jnp.dot(h, w2_ref[:, :], preferred_element_type=f32)
                    + b2_ref[:, :], 0.0)
    h = jnp.maximum(jnp.dot(h, w3_ref[:, :], preferred_element_type=f32)
                    + b3_ref[:, :], 0.0)
    jrow = (jax.lax.broadcasted_iota(jnp.int32, (ns * M, 1), 0)
            // M).astype(f32)
    countrep = jnp.concatenate([count] * ns, axis=0)
    h = jnp.where(countrep > jrow, h, -jnp.inf)
    out_ref[:, :] = jnp.max(h.reshape(ns, M, h.shape[1]), axis=0)


def _sa_scale_pallas(centers8, pointsT, pxf, layers, ns, radius):
    npoint = centers8.shape[0]
    N = pointsT.shape[1]
    (W1, b1), (W2, b2), (W3, b3) = _bn_fold(layers)
    C1, C2, C3 = W1.shape[1], W2.shape[1], W3.shape[1]
    W1p = jnp.zeros((8, C1), jnp.float32).at[0:3, :].set(W1[0:3, :])
    M = min(128, npoint)
    grid = npoint // M
    body = functools.partial(_sa_scale_body, ns=ns, r2=float(radius * radius),
                             M=M, N=N)
    full = lambda arr: pl.BlockSpec(arr.shape, lambda i: (0, 0))
    out = pl.pallas_call(
        body,
        grid=(grid,),
        in_specs=[
            pl.BlockSpec((M, 8), lambda i: (i, 0)),
            full(pointsT), full(pxf), full(W1), full(W1p),
            full(b1.reshape(1, -1)), full(W2), full(b2.reshape(1, -1)),
            full(W3), full(b3.reshape(1, -1)),
        ],
        out_specs=pl.BlockSpec((M, C3), lambda i: (i, 0)),
        out_shape=jax.ShapeDtypeStruct((npoint, C3), jnp.float32),
    )(centers8, pointsT, pxf, W1, W1p, b1.reshape(1, -1), W2,
      b2.reshape(1, -1), W3, b3.reshape(1, -1))
    return out


def _bn_fold(layers):
    out = []
    for (W, b, g, beta) in layers:
        s = g / jnp.sqrt(1.0 + 1e-5)
        out.append((W * s[None, :], b * s + beta))
    return out


def _sqdist2(src, dst):
    d = -2.0 * jnp.matmul(src, dst.T)
    d = d + jnp.sum(src * src, axis=-1)[:, None]
    d = d + jnp.sum(dst * dst, axis=-1)[None, :]
    return d


def _ball_query2(radius, nsample, xyz, new_xyz):
    N = xyz.shape[0]
    sqd = _sqdist2(new_xyz, xyz)
    mask = sqd <= radius * radius
    scores = jnp.where(mask, -jnp.arange(N, dtype=jnp.float32), -jnp.inf)
    vals, idx = jax.lax.top_k(scores, nsample)
    valid = vals > -jnp.inf
    idx = jnp.where(valid, idx, idx[:, 0:1])
    return idx


def _sa_msg2(xyz, features, npoint, radii, nsamples, scale_params):
    new_xyz = _fps_pallas(xyz, npoint)
    N = xyz.shape[0]
    centers8 = jnp.zeros((npoint, 8), jnp.float32).at[:, 0:3].set(new_xyz)
    pointsT = jnp.zeros((8, N), jnp.float32).at[0:3, :].set(xyz.T)
    pxf = jnp.concatenate([xyz, features.T], axis=1)
    outs = []
    for radius, ns, layers in zip(radii, nsamples, scale_params):
        outs.append(_sa_scale_pallas(centers8, pointsT, pxf, layers, ns,
                                     radius))
    return new_xyz, jnp.concatenate(outs, axis=-1).T


def _fp2(unknown, known, unknown_feats, known_feats, layers):
    sqd = _sqdist2(unknown, known)
    vals, idx = jax.lax.top_k(-sqd, 3)
    dist = -vals
    dist_recip = 1.0 / (dist + 1e-8)
    weight = dist_recip / jnp.sum(dist_recip, axis=1, keepdims=True)
    kf = known_feats.T
    interp = jnp.sum(kf[idx] * weight[..., None], axis=1)
    if unknown_feats is not None:
        new = jnp.concatenate([interp, unknown_feats.T], axis=-1)
    else:
        new = interp
    h = new
    for (W, b) in _bn_fold(layers):
        h = jax.nn.relu(jnp.matmul(h, W) + b)
    return h.T


def kernel(pointcloud, params):
    xyz = pointcloud[0, :, 0:3]
    features = pointcloud[0, :, 3:].T
    l_xyz = [xyz]
    l_feats = [features]
    for k in range(4):
        npoint, radii, nsamples, _ = _SA_CFG[k]
        nx, nf = _sa_msg2(l_xyz[k], l_feats[k], npoint, radii, nsamples,
                          params["sa"][k])
        l_xyz.append(nx)
        l_feats.append(nf)
    for i in range(-1, -5, -1):
        l_feats[i - 1] = _fp2(l_xyz[i - 1], l_xyz[i], l_feats[i - 1],
                              l_feats[i], params["fp"][i])
    point_features = l_feats[0]
    return jnp.max(point_features, axis=-1)


# speedup vs baseline: 19.4526x; 1.3730x over previous
import functools

import jax
import jax.numpy as jnp
import numpy as np
from jax.experimental import pallas as pl
from jax.experimental.pallas import tpu as pltpu

_SA_CFG = [
    (4096, [0.1, 0.5], [16, 32], [[4, 16, 16, 32], [4, 32, 32, 64]]),
    (1024, [0.5, 1.0], [16, 32], [[99, 64, 64, 128], [99, 64, 96, 128]]),
    (256, [1.0, 2.0], [16, 32], [[259, 128, 196, 256], [259, 128, 196, 256]]),
    (64, [2.0, 4.0], [16, 32], [[515, 256, 256, 512], [515, 256, 384, 512]]),
]


def _fps_kernel_body(xs_ref, ys_ref, zs_ref, ox_ref, oy_ref, oz_ref, dist_ref,
                     *, npoint: int):
    S, L = xs_ref.shape
    N = S * L
    pos = (jax.lax.broadcasted_iota(jnp.int32, (S, L), 0) * L
           + jax.lax.broadcasted_iota(jnp.int32, (S, L), 1))
    dist_ref[:, :] = jnp.full((S, L), 1e10, jnp.float32)

    def body(i, far):
        m = pos == far
        cx = jnp.sum(jnp.where(m, xs_ref[:, :], 0.0))
        cy = jnp.sum(jnp.where(m, ys_ref[:, :], 0.0))
        cz = jnp.sum(jnp.where(m, zs_ref[:, :], 0.0))
        ox_ref[pl.ds(i, 1), :] = jnp.full((1, 1), cx, jnp.float32)
        oy_ref[pl.ds(i, 1), :] = jnp.full((1, 1), cy, jnp.float32)
        oz_ref[pl.ds(i, 1), :] = jnp.full((1, 1), cz, jnp.float32)
        dx = xs_ref[:, :] - cx
        dy = ys_ref[:, :] - cy
        dz = zs_ref[:, :] - cz
        d = dx * dx + dy * dy + dz * dz
        nd = jnp.minimum(dist_ref[:, :], d)
        dist_ref[:, :] = nd
        mx = jnp.max(nd)
        far2 = jnp.min(jnp.where(nd == mx, pos, N)).astype(jnp.int32)
        return far2

    jax.lax.fori_loop(0, npoint, body, jnp.int32(0), unroll=False)


def _fps_pallas(xyz, npoint):
    N = xyz.shape[0]
    S = 8
    L = N // S
    xs = xyz[:, 0].reshape(S, L)
    ys = xyz[:, 1].reshape(S, L)
    zs = xyz[:, 2].reshape(S, L)
    out_sds = jax.ShapeDtypeStruct((npoint, 1), jnp.float32)
    ox, oy, oz = pl.pallas_call(
        functools.partial(_fps_kernel_body, npoint=npoint),
        out_shape=(out_sds, out_sds, out_sds),
        scratch_shapes=[pltpu.VMEM((S, L), jnp.float32)],
    )(xs, ys, zs)
    return jnp.concatenate([ox, oy, oz], axis=1)


def _sa_scale_body(centers_ref, pointsT_ref, pxf_ref, w1_ref, w1p_ref,
                   b1_ref, w2_ref, b2_ref, w3_ref, b3_ref, out_ref,
                   *, ns: int, r2: float, M: int, N: int):
    f32 = jnp.float32
    t = jnp.dot(pxf_ref[:, :], w1_ref[:, :],
                preferred_element_type=f32) + b1_ref[:, :]
    cblk = centers_ref[:, :]
    pT = pointsT_ref[:, :]
    dots = jnp.dot(cblk, pT, preferred_element_type=f32)
    cn2 = jnp.sum(cblk * cblk, axis=1, keepdims=True)
    pn2 = jnp.sum(pT * pT, axis=0, keepdims=True)
    sqd = (-2.0 * dots + cn2) + pn2
    m01 = (sqd <= r2).astype(f32)
    triu = (jax.lax.broadcasted_iota(jnp.int32, (128, 128), 0)
            <= jax.lax.broadcasted_iota(jnp.int32, (128, 128), 1)).astype(f32)
    parts = []
    carry = jnp.zeros((M, 1), f32)
    for b in range(N // 128):
        rb = jnp.dot(m01[:, b * 128:(b + 1) * 128], triu,
                     preferred_element_type=f32) + carry
        parts.append(rb)
        carry = rb[:, 127:128]
    rank = jnp.concatenate(parts, axis=1)
    count = carry
    s = jnp.where((m01 > 0.0) & (rank <= float(ns)), rank, 0.0)
    gs = []
    for j in range(ns):
        ohj = (s == float(j + 1)).astype(f32)
        gs.append(jnp.dot(ohj, t, preferred_element_type=f32))
    g = jnp.concatenate(gs, axis=0)
    u = jnp.dot(cblk, w1p_ref[:, :], preferred_element_type=f32)
    urep = jnp.concatenate([u] * ns, axis=0)
    h = jnp.maximum(g - urep, 0.0)
    h = jnp.maximum(jnp.dot(h, w2_ref[:, :], preferred_element_type=f32)
                    + b2_ref[:, :], 0.0)
    h = jnp.maximum(jnp.dot(h, w3_ref[:, :], preferred_element_type=f32)
                    + b3_ref[:, :], 0.0)
    jrow = (jax.lax.broadcasted_iota(jnp.int32, (ns * M, 1), 0)
            // M).astype(f32)
    countrep = jnp.concatenate([count] * ns, axis=0)
    h = jnp.where(countrep > jrow, h, -jnp.inf)
    out_ref[:, :] = jnp.max(h.reshape(ns, M, h.shape[1]), axis=0)


def _sa_scale_pallas(centers8, pointsT, pxf, layers, ns, radius):
    npoint = centers8.shape[0]
    N = pointsT.shape[1]
    (W1, b1), (W2, b2), (W3, b3) = _bn_fold(layers)
    C1, C2, C3 = W1.shape[1], W2.shape[1], W3.shape[1]
    W1p = jnp.zeros((8, C1), jnp.float32).at[0:3, :].set(W1[0:3, :])
    M = min(128, npoint)
    grid = npoint // M
    body = functools.partial(_sa_scale_body, ns=ns, r2=float(radius * radius),
                             M=M, N=N)
    full = lambda arr: pl.BlockSpec(arr.shape, lambda i: (0, 0))
    out = pl.pallas_call(
        body,
        grid=(grid,),
        in_specs=[
            pl.BlockSpec((M, 8), lambda i: (i, 0)),
            full(pointsT), full(pxf), full(W1), full(W1p),
            full(b1.reshape(1, -1)), full(W2), full(b2.reshape(1, -1)),
            full(W3), full(b3.reshape(1, -1)),
        ],
        out_specs=pl.BlockSpec((M, C3), lambda i: (i, 0)),
        out_shape=jax.ShapeDtypeStruct((npoint, C3), jnp.float32),
    )(centers8, pointsT, pxf, W1, W1p, b1.reshape(1, -1), W2,
      b2.reshape(1, -1), W3, b3.reshape(1, -1))
    return out


def _fp_body(unk_ref, knownT_ref, kf_ref, uf_ref, w1_ref, b1_ref, w2_ref,
             b2_ref, out_ref, *, m: int, M: int, final_max: bool):
    f32 = jnp.float32
    ublk = unk_ref[:, :]
    kT = knownT_ref[:, :]
    dots = jnp.dot(ublk, kT, preferred_element_type=f32)
    un2 = jnp.sum(ublk * ublk, axis=1, keepdims=True)
    kn2 = jnp.sum(kT * kT, axis=0, keepdims=True)
    sqd = (-2.0 * dots + un2) + kn2
    lane = jax.lax.broadcasted_iota(jnp.int32, (M, m), 1).astype(f32)
    d = sqd
    iks, mns = [], []
    for _ in range(3):
        mn = jnp.min(d, axis=1, keepdims=True)
        ik = jnp.min(jnp.where(d == mn, lane, float(m)), axis=1,
                     keepdims=True)
        iks.append(ik)
        mns.append(mn)
        d = jnp.where(lane == ik, jnp.inf, d)
    recips = [1.0 / (mn + 1e-8) for mn in mns]
    tot = recips[0] + recips[1] + recips[2]
    wmat = sum(((lane == ik).astype(f32) * (rc / tot)
                for ik, rc in zip(iks, recips)), jnp.zeros((M, m), f32))
    interp = jnp.dot(wmat, kf_ref[:, :], preferred_element_type=f32)
    new = jnp.concatenate([interp, uf_ref[:, :]], axis=1)
    h = jnp.maximum(jnp.dot(new, w1_ref[:, :], preferred_element_type=f32)
                    + b1_ref[:, :], 0.0)
    h = jnp.maximum(jnp.dot(h, w2_ref[:, :], preferred_element_type=f32)
                    + b2_ref[:, :], 0.0)
    if final_max:
        i = pl.program_id(0)

        @pl.when(i == 0)
        def _():
            out_ref[:, :] = jnp.full(out_ref.shape, -jnp.inf, f32)

        out_ref[:, :] = jnp.maximum(out_ref[:, :],
                                    jnp.max(h, axis=0, keepdims=True))
    else:
        out_ref[:, :] = h


def _fp_pallas(unknown, known, unknown_featsT, known_featsT, layers,
               final_max=False):
    n, m = unknown.shape[0], known.shape[0]
    (W1, b1), (W2, b2) = _bn_fold(layers)
    Cout = W2.shape[1]
    unk8 = jnp.zeros((n, 8), jnp.float32).at[:, 0:3].set(unknown)
    knT = jnp.zeros((8, m), jnp.float32).at[0:3, :].set(known.T)
    M = min(128, n)
    grid = n // M
    body = functools.partial(_fp_body, m=m, M=M, final_max=final_max)
    full = lambda arr: pl.BlockSpec(arr.shape, lambda i: (0, 0))
    if final_max:
        out_spec = pl.BlockSpec((1, Cout), lambda i: (0, 0))
        out_shape = jax.ShapeDtypeStruct((1, Cout), jnp.float32)
    else:
        out_spec = pl.BlockSpec((M, Cout), lambda i: (i, 0))
        out_shape = jax.ShapeDtypeStruct((n, Cout), jnp.float32)
    return pl.pallas_call(
        body,
        grid=(grid,),
        in_specs=[
            pl.BlockSpec((M, 8), lambda i: (i, 0)),
            full(knT), full(known_featsT),
            pl.BlockSpec((M, unknown_featsT.shape[1]), lambda i: (i, 0)),
            full(W1), full(b1.reshape(1, -1)), full(W2),
            full(b2.reshape(1, -1)),
        ],
        out_specs=out_spec,
        out_shape=out_shape,
    )(unk8, knT, known_featsT, unknown_featsT, W1, b1.reshape(1, -1),
      W2, b2.reshape(1, -1))


def _bn_fold(layers):
    out = []
    for (W, b, g, beta) in layers:
        s = g / jnp.sqrt(1.0 + 1e-5)
        out.append((W * s[None, :], b * s + beta))
    return out


def _sa_msg2(xyz, featsT, npoint, radii, nsamples, scale_params):
    new_xyz = _fps_pallas(xyz, npoint)
    N = xyz.shape[0]
    centers8 = jnp.zeros((npoint, 8), jnp.float32).at[:, 0:3].set(new_xyz)
    pointsT = jnp.zeros((8, N), jnp.float32).at[0:3, :].set(xyz.T)
    pxf = jnp.concatenate([xyz, featsT], axis=1)
    outs = []
    for radius, ns, layers in zip(radii, nsamples, scale_params):
        outs.append(_sa_scale_pallas(centers8, pointsT, pxf, layers, ns,
                                     radius))
    return new_xyz, jnp.concatenate(outs, axis=-1)


def kernel(pointcloud, params):
    xyz = pointcloud[0, :, 0:3]
    featsT = pointcloud[0, :, 3:]
    l_xyz = [xyz]
    l_feats = [featsT]
    for k in range(4):
        npoint, radii, nsamples, _ = _SA_CFG[k]
        nx, nf = _sa_msg2(l_xyz[k], l_feats[k], npoint, radii, nsamples,
                          params["sa"][k])
        l_xyz.append(nx)
        l_feats.append(nf)
    for i in range(-1, -5, -1):
        final = i == -4
        res = _fp_pallas(l_xyz[i - 1], l_xyz[i], l_feats[i - 1], l_feats[i],
                         params["fp"][i], final_max=final)
        l_feats[i - 1] = res
    return l_feats[0].reshape(-1)


# speedup vs baseline: 27.8826x; 1.4334x over previous
import functools

import jax
import jax.numpy as jnp
import numpy as np
from jax.experimental import pallas as pl
from jax.experimental.pallas import tpu as pltpu

_SA_CFG = [
    (4096, [0.1, 0.5], [16, 32], [[4, 16, 16, 32], [4, 32, 32, 64]]),
    (1024, [0.5, 1.0], [16, 32], [[99, 64, 64, 128], [99, 64, 96, 128]]),
    (256, [1.0, 2.0], [16, 32], [[259, 128, 196, 256], [259, 128, 196, 256]]),
    (64, [2.0, 4.0], [16, 32], [[515, 256, 256, 512], [515, 256, 384, 512]]),
]


def _fps_kernel_body(xyzr_ref, xs_ref, ys_ref, zs_ref, xr_ref, yr_ref, zr_ref,
                     onew_ref, dist_ref, *, npoint: int):
    S, L = xs_ref.shape
    N = S * L
    posf = (jax.lax.broadcasted_iota(jnp.int32, (S, L), 0) * L
            + jax.lax.broadcasted_iota(jnp.int32, (S, L), 1)).astype(
                jnp.float32)
    dist_ref[:, :] = jnp.full((S, L), 1e10, jnp.float32)

    def body(i, far):
        onew_ref[pl.ds(i, 1), :] = xyzr_ref[pl.ds(far, 1), :]
        cx = jnp.broadcast_to(xr_ref[pl.ds(far, 1), :], (S, L))
        cy = jnp.broadcast_to(yr_ref[pl.ds(far, 1), :], (S, L))
        cz = jnp.broadcast_to(zr_ref[pl.ds(far, 1), :], (S, L))
        dx = xs_ref[:, :] - cx
        dy = ys_ref[:, :] - cy
        dz = zs_ref[:, :] - cz
        d = dx * dx + dy * dy + dz * dz
        nd = jnp.minimum(dist_ref[:, :], d)
        dist_ref[:, :] = nd
        v, ix = nd, posf
        k = S
        while k > 1:
            k //= 2
            va, vb = v[0:k], v[k:2 * k]
            ia, ib = ix[0:k], ix[k:2 * k]
            take = vb > va
            v = jnp.where(take, vb, va)
            ix = jnp.where(take, ib, ia)
        m = jnp.max(v, axis=1, keepdims=True)
        sel = jnp.where(v == m, ix, float(N))
        far2 = jnp.min(sel).astype(jnp.int32)
        return far2

    jax.lax.fori_loop(0, npoint, body, jnp.int32(0), unroll=False)


def _fps_pallas(xyz, npoint):
    N = xyz.shape[0]
    S = N // 128
    L = 128
    xs = xyz[:, 0].reshape(S, L)
    ys = xyz[:, 1].reshape(S, L)
    zs = xyz[:, 2].reshape(S, L)
    xr = jnp.broadcast_to(xyz[:, 0:1], (N, 128))
    yr = jnp.broadcast_to(xyz[:, 1:2], (N, 128))
    zr = jnp.broadcast_to(xyz[:, 2:3], (N, 128))
    return pl.pallas_call(
        functools.partial(_fps_kernel_body, npoint=npoint),
        out_shape=jax.ShapeDtypeStruct((npoint, 3), jnp.float32),
        scratch_shapes=[pltpu.VMEM((S, L), jnp.float32)],
    )(xyz, xs, ys, zs, xr, yr, zr)


def _sa_scale_body(centers_ref, pointsT_ref, pxf_ref, w1_ref, w1p_ref,
                   b1_ref, w2_ref, b2_ref, w3_ref, b3_ref, out_ref,
                   *, ns: int, r2: float, M: int, N: int):
    f32 = jnp.float32
    t = jnp.dot(pxf_ref[:, :], w1_ref[:, :],
                preferred_element_type=f32) + b1_ref[:, :]
    cblk = centers_ref[:, :]
    pT = pointsT_ref[:, :]
    dots = jnp.dot(cblk, pT, preferred_element_type=f32)
    cn2 = jnp.sum(cblk * cblk, axis=1, keepdims=True)
    pn2 = jnp.sum(pT * pT, axis=0, keepdims=True)
    sqd = (-2.0 * dots + cn2) + pn2
    m01 = (sqd <= r2).astype(f32)
    triu = (jax.lax.broadcasted_iota(jnp.int32, (128, 128), 0)
            <= jax.lax.broadcasted_iota(jnp.int32, (128, 128), 1)).astype(f32)
    parts = []
    carry = jnp.zeros((M, 1), f32)
    for b in range(N // 128):
        rb = jnp.dot(m01[:, b * 128:(b + 1) * 128], triu,
                     preferred_element_type=f32) + carry
        parts.append(rb)
        carry = rb[:, 127:128]
    rank = jnp.concatenate(parts, axis=1)
    count = carry
    s = jnp.where((m01 > 0.0) & (rank <= float(ns)), rank, 0.0)
    gs = []
    for j in range(ns):
        ohj = (s == float(j + 1)).astype(f32)
        gs.append(jnp.dot(ohj, t, preferred_element_type=f32))
    g = jnp.concatenate(gs, axis=0)
    u = jnp.dot(cblk, w1p_ref[:, :], preferred_element_type=f32)
    urep = jnp.concatenate([u] * ns, axis=0)
    h = jnp.maximum(g - urep, 0.0)
    h = jnp.maximum(jnp.dot(h, w2_ref[:, :], preferred_element_type=f32)
                    + b2_ref[:, :], 0.0)
    h = jnp.maximum(jnp.dot(h, w3_ref[:, :], preferred_element_type=f32)
                    + b3_ref[:, :], 0.0)
    jrow = (jax.lax.broadcasted_iota(jnp.int32, (ns * M, 1), 0)
            // M).astype(f32)
    countrep = jnp.concatenate([count] * ns, axis=0)
    h = jnp.where(countrep > jrow, h, -jnp.inf)
    out_ref[:, :] = jnp.max(h.reshape(ns, M, h.shape[1]), axis=0)


def _sa_scale_pallas(centers8, pointsT, pxf, layers, ns, radius):
    npoint = centers8.shape[0]
    N = pointsT.shape[1]
    (W1, b1), (W2, b2), (W3, b3) = _bn_fold(layers)
    C1, C2, C3 = W1.shape[1], W2.shape[1], W3.shape[1]
    W1p = jnp.zeros((8, C1), jnp.float32).at[0:3, :].set(W1[0:3, :])
    M = min(128, npoint)
    grid = npoint // M
    body = functools.partial(_sa_scale_body, ns=ns, r2=float(radius * radius),
                             M=M, N=N)
    full = lambda arr: pl.BlockSpec(arr.shape, lambda i: (0, 0))
    out = pl.pallas_call(
        body,
        grid=(grid,),
        in_specs=[
            pl.BlockSpec((M, 8), lambda i: (i, 0)),
            full(pointsT), full(pxf), full(W1), full(W1p),
            full(b1.reshape(1, -1)), full(W2), full(b2.reshape(1, -1)),
            full(W3), full(b3.reshape(1, -1)),
        ],
        out_specs=pl.BlockSpec((M, C3), lambda i: (i, 0)),
        out_shape=jax.ShapeDtypeStruct((npoint, C3), jnp.float32),
    )(centers8, pointsT, pxf, W1, W1p, b1.reshape(1, -1), W2,
      b2.reshape(1, -1), W3, b3.reshape(1, -1))
    return out


def _fp_body(unk_ref, knownT_ref, kf_ref, uf_ref, w1_ref, b1_ref, w2_ref,
             b2_ref, out_ref, *, m: int, M: int, final_max: bool):
    f32 = jnp.float32
    ublk = unk_ref[:, :]
    kT = knownT_ref[:, :]
    dots = jnp.dot(ublk, kT, preferred_element_type=f32)
    un2 = jnp.sum(ublk * ublk, axis=1, keepdims=True)
    kn2 = jnp.sum(kT * kT, axis=0, keepdims=True)
    sqd = (-2.0 * dots + un2) + kn2
    lane = jax.lax.broadcasted_iota(jnp.int32, (M, m), 1).astype(f32)
    d = sqd
    iks, mns = [], []
    for _ in range(3):
        mn = jnp.min(d, axis=1, keepdims=True)
        ik = jnp.min(jnp.where(d == mn, lane, float(m)), axis=1,
                     keepdims=True)
        iks.append(ik)
        mns.append(mn)
        d = jnp.where(lane == ik, jnp.inf, d)
    recips = [1.0 / (mn + 1e-8) for mn in mns]
    tot = recips[0] + recips[1] + recips[2]
    wmat = sum(((lane == ik).astype(f32) * (rc / tot)
                for ik, rc in zip(iks, recips)), jnp.zeros((M, m), f32))
    interp = jnp.dot(wmat, kf_ref[:, :], preferred_element_type=f32)
    new = jnp.concatenate([interp, uf_ref[:, :]], axis=1)
    h = jnp.maximum(jnp.dot(new, w1_ref[:, :], preferred_element_type=f32)
                    + b1_ref[:, :], 0.0)
    h = jnp.maximum(jnp.dot(h, w2_ref[:, :], preferred_element_type=f32)
                    + b2_ref[:, :], 0.0)
    if final_max:
        i = pl.program_id(0)

        @pl.when(i == 0)
        def _():
            out_ref[:, :] = jnp.full(out_ref.shape, -jnp.inf, f32)

        out_ref[:, :] = jnp.maximum(out_ref[:, :],
                                    jnp.max(h, axis=0, keepdims=True))
    else:
        out_ref[:, :] = h


def _fp_pallas(unknown, known, unknown_featsT, known_featsT, layers,
               final_max=False):
    n, m = unknown.shape[0], known.shape[0]
    (W1, b1), (W2, b2) = _bn_fold(layers)
    Cout = W2.shape[1]
    unk8 = jnp.zeros((n, 8), jnp.float32).at[:, 0:3].set(unknown)
    knT = jnp.zeros((8, m), jnp.float32).at[0:3, :].set(known.T)
    M = min(128, n)
    grid = n // M
    body = functools.partial(_fp_body, m=m, M=M, final_max=final_max)
    full = lambda arr: pl.BlockSpec(arr.shape, lambda i: (0, 0))
    if final_max:
        out_spec = pl.BlockSpec((1, Cout), lambda i: (0, 0))
        out_shape = jax.ShapeDtypeStruct((1, Cout), jnp.float32)
    else:
        out_spec = pl.BlockSpec((M, Cout), lambda i: (i, 0))
        out_shape = jax.ShapeDtypeStruct((n, Cout), jnp.float32)
    return pl.pallas_call(
        body,
        grid=(grid,),
        in_specs=[
            pl.BlockSpec((M, 8), lambda i: (i, 0)),
            full(knT), full(known_featsT),
            pl.BlockSpec((M, unknown_featsT.shape[1]), lambda i: (i, 0)),
            full(W1), full(b1.reshape(1, -1)), full(W2),
            full(b2.reshape(1, -1)),
        ],
        out_specs=out_spec,
        out_shape=out_shape,
    )(unk8, knT, known_featsT, unknown_featsT, W1, b1.reshape(1, -1),
      W2, b2.reshape(1, -1))


def _bn_fold(layers):
    out = []
    for (W, b, g, beta) in layers:
        s = g / jnp.sqrt(1.0 + 1e-5)
        out.append((W * s[None, :], b * s + beta))
    return out


def _sa_msg2(xyz, featsT, npoint, radii, nsamples, scale_params):
    new_xyz = _fps_pallas(xyz, npoint)
    N = xyz.shape[0]
    centers8 = jnp.zeros((npoint, 8), jnp.float32).at[:, 0:3].set(new_xyz)
    pointsT = jnp.zeros((8, N), jnp.float32).at[0:3, :].set(xyz.T)
    pxf = jnp.concatenate([xyz, featsT], axis=1)
    outs = []
    for radius, ns, layers in zip(radii, nsamples, scale_params):
        outs.append(_sa_scale_pallas(centers8, pointsT, pxf, layers, ns,
                                     radius))
    return new_xyz, jnp.concatenate(outs, axis=-1)


def kernel(pointcloud, params):
    xyz = pointcloud[0, :, 0:3]
    featsT = pointcloud[0, :, 3:]
    l_xyz = [xyz]
    l_feats = [featsT]
    for k in range(4):
        npoint, radii, nsamples, _ = _SA_CFG[k]
        nx, nf = _sa_msg2(l_xyz[k], l_feats[k], npoint, radii, nsamples,
                          params["sa"][k])
        l_xyz.append(nx)
        l_feats.append(nf)
    for i in range(-1, -5, -1):
        final = i == -4
        res = _fp_pallas(l_xyz[i - 1], l_xyz[i], l_feats[i - 1], l_feats[i],
                         params["fp"][i], final_max=final)
        l_feats[i - 1] = res
    return l_feats[0].reshape(-1)
